# baseline (speedup 1.0000x reference)
import functools

import jax
import jax.numpy as jnp
from jax import lax
from jax.experimental import pallas as pl
from jax.experimental.pallas import tpu as pltpu
from jax.experimental.pallas import tpu_sc as plsc

N = 1024
NSL = N // 16
SENT = 1 << 20


def _prologue_body(logits_ref, u_ref, adj_ref, gl_ref, explog_ref, deg_ref,
                   bits_ref, itc_ref):
    l = logits_ref[...]
    u = u_ref[...]
    gn = -jnp.log(-jnp.log(u + 1e-20) + 1e-20)
    gl_ref[...] = l + gn
    explog_ref[...] = jnp.exp(l)
    a = adj_ref[...]
    deg_ref[...] = jnp.sum(a, axis=1, keepdims=True)
    acc = jnp.zeros((32, N), jnp.int32)
    for b in range(32):
        acc = acc | (lax.slice(a, (b * 32, 0), (b * 32 + 32, N)) << b)
    bits_ref[...] = acc
    srl = lax.shift_right_logical
    x = acc - (srl(acc, 1) & 0x55555555)
    x = (x & 0x33333333) + (srl(x, 2) & 0x33333333)
    x = (x + srl(x, 4)) & 0x0F0F0F0F
    cnt = srl(x * 0x01010101, 24)
    itc_ref[...] = jnp.max(cnt, axis=0, keepdims=True)


_prologue = pl.pallas_call(
    _prologue_body,
    out_shape=(
        jax.ShapeDtypeStruct((8, 128), jnp.float32),
        jax.ShapeDtypeStruct((8, 128), jnp.float32),
        jax.ShapeDtypeStruct((N, 1), jnp.int32),
        jax.ShapeDtypeStruct((32, N), jnp.int32),
        jax.ShapeDtypeStruct((1, N), jnp.int32),
    ),
)


def _epilogue_body(logits_ref, sp_ref, out_ref):
    l = logits_ref[...]
    s = jnp.sum(sp_ref[...], axis=1, keepdims=True)
    out_ref[...] = 0.0 - jnp.log(s * jnp.exp(-l) + 1e-10)


_epilogue = pl.pallas_call(
    _epilogue_body,
    out_shape=jax.ShapeDtypeStruct((N, 1), jnp.float32),
)


def _make_sc_kernel():
    mesh = plsc.VectorSubcoreMesh(core_axis_name="c", subcore_axis_name="s")

    @functools.partial(
        pl.kernel,
        mesh=mesh,
        out_type=jax.ShapeDtypeStruct((N * 16,), jnp.float32),
        compiler_params=pltpu.CompilerParams(needs_layout_passes=False),
        scratch_types=[
            pltpu.VMEM((N,), jnp.float32),
            pltpu.VMEM((64,), jnp.float32),
            pltpu.VMEM((64,), jnp.float32),
            pltpu.VMEM((64,), jnp.int32),
            pltpu.VMEM((N,), jnp.int32),
            pltpu.VMEM((32, N), jnp.int32),
            pltpu.VMEM((N,), jnp.float32),
            pltpu.VMEM((N,), jnp.float32),
            pltpu.VMEM((N,), jnp.int32),
            pltpu.VMEM((N * 16,), jnp.float32),
        ],
    )
    def sc_kernel(gl_hbm, explog_hbm, deg_hbm, bits_hbm, itc_hbm, sp_hbm,
                  mval_v, bmax_v, bsum_v, bidx_v, deg_v, bits_v, gl_v,
                  explog_v, itc_v, sp_v):
        c = lax.axis_index("c")
        sub = lax.axis_index("s")

        def _perm(v, idx):
            return v.at[idx].get(mode="promise_in_bounds")

        def _splat(x):
            return x if getattr(x, "shape", ()) == (16,) else jnp.full(
                (16,), x, jnp.int32)

        def _bcast_last(v):
            return _perm(v, jnp.full((16,), 15, jnp.int32))

        @pl.when(jnp.logical_and(c == 0, sub == 0))
        def _():
            pltpu.sync_copy(gl_hbm, gl_v)
            pltpu.sync_copy(explog_hbm, explog_v)
            pltpu.sync_copy(deg_hbm, deg_v)
            pltpu.sync_copy(bits_hbm, bits_v)
            pltpu.sync_copy(itc_hbm, itc_v)
            iota = lax.broadcasted_iota(jnp.int32, (16,), 0)
            neg_inf = jnp.full((16,), -jnp.inf, jnp.float32)
            zeros_f = jnp.zeros((16,), jnp.float32)
            lane0 = iota == 0

            for k in range(NSL):
                sl = pl.ds(k * 16, 16)
                elig = deg_v[sl] == 0
                mval_v[sl] = jnp.where(elig, gl_v[sl], neg_inf)
            for v in range(4):
                bm = neg_inf
                bs = zeros_f
                bi = jnp.zeros((16,), jnp.int32)
                for t in range(16):
                    sl = pl.ds(t * 64 + v * 16, 16)
                    mv = mval_v[sl]
                    m = mv > bm
                    bm = jnp.where(m, mv, bm)
                    bi = jnp.where(m, t, bi)
                    bs = bs + jnp.where(mv > neg_inf, explog_v[sl], zeros_f)
                bmax_v[pl.ds(v * 16, 16)] = bm
                bsum_v[pl.ds(v * 16, 16)] = bs
                bidx_v[pl.ds(v * 16, 16)] = bi

            def step(_, carry):
                bb = bmax_v[pl.ds(0, 16)]
                bid = jnp.zeros((16,), jnp.int32)
                ss = bsum_v[pl.ds(0, 16)]
                for v in range(1, 4):
                    bv = bmax_v[pl.ds(v * 16, 16)]
                    m = bv > bb
                    bb = jnp.where(m, bv, bb)
                    bid = jnp.where(m, v, bid)
                    ss = ss + bsum_v[pl.ds(v * 16, 16)]
                r = _bcast_last(plsc.cummax(bb))
                lane = _splat(plsc.all_reduce_ffs(bb == r))
                vid = _perm(bid, lane)
                blk = vid * 16 + lane

                t = plsc.load_gather(bidx_v, [blk])
                mi_v = blk + 64 * t

                plsc.store_scatter(sp_v, [mi_v * 16 + iota], ss)
                plsc.store_scatter(mval_v, [mi_v], neg_inf, mask=lane0)
                plsc.store_scatter(deg_v, [mi_v],
                                   jnp.full((16,), SENT, jnp.int32),
                                   mask=lane0)
                idxb = blk + 64 * iota
                bvals = plsc.load_gather(mval_v, [idxb])
                bv2 = jnp.where(iota == t, neg_inf, bvals)
                nbm = _bcast_last(plsc.cummax(bv2))
                nt = _splat(plsc.all_reduce_ffs(bv2 == nbm))
                plsc.store_scatter(bmax_v, [blk], nbm, mask=lane0)
                plsc.store_scatter(bidx_v, [blk], nt, mask=lane0)
                eg = plsc.load_gather(explog_v, [mi_v])
                plsc.addupdate_scatter(bsum_v, [blk], -eg, mask=lane0)

                w0 = plsc.load_gather(bits_v, [iota, mi_v])
                w1 = plsc.load_gather(bits_v, [iota + 16, mi_v])
                trip = jnp.max(plsc.load_gather(itc_v, [mi_v]))

                def adj_iter(wc):
                    nw = []
                    for wv, rowidx in ((wc[0], iota), (wc[1], iota + 16)):
                        act = wv != 0
                        y = wv & (-wv)
                        f = y.astype(jnp.float32)
                        e = ((lax.bitcast_convert_type(f, jnp.int32) >> 23)
                             & 255) - 127
                        i = jnp.where(act, e * 32 + rowidx, 0)
                        blkv = i & 63
                        g = plsc.load_gather(gl_v, [i], mask=act)
                        ee = plsc.load_gather(explog_v, [i], mask=act)
                        bm = plsc.load_gather(bmax_v, [blkv], mask=act)
                        od = plsc.load_gather(deg_v, [i], mask=act)
                        nd = od - 1
                        plsc.store_scatter(deg_v, [i], nd, mask=act)
                        trans = jnp.logical_and(act, nd == 0)
                        plsc.store_scatter(mval_v, [i], g, mask=trans)
                        plsc.store_scatter(bmax_v, [blkv], jnp.maximum(bm, g),
                                           mask=trans)
                        newmax = jnp.logical_and(trans, g > bm)
                        plsc.store_scatter(bidx_v, [blkv],
                                           lax.shift_right_logical(i, 6),
                                           mask=newmax)
                        plsc.addupdate_scatter(bsum_v, [blkv], ee, mask=trans)
                        nw.append(wv & (wv - 1))
                    return tuple(nw)

                wc = adj_iter((w0, w1))

                def wcond(c):
                    return c[2] > 1

                def wbody(c):
                    nw = adj_iter((c[0], c[1]))
                    return (nw[0], nw[1], c[2] - 1)

                lax.while_loop(wcond, wbody, (wc[0], wc[1], trip))
                return carry

            lax.fori_loop(0, N, step, 0)
            pltpu.sync_copy(sp_v, sp_hbm)

    return sc_kernel


_sc_kernel = _make_sc_kernel()


def kernel(logits, adj_mat):
    u = jax.random.uniform(jax.random.key(42), logits.shape,
                           dtype=logits.dtype)
    l2 = logits.reshape(8, 128)
    gl2, explog2, deg2, bits, itc = _prologue(l2, u.reshape(8, 128), adj_mat)
    gl = gl2.reshape(N)
    sp = _sc_kernel(gl, explog2.reshape(N), deg2.reshape(N), bits,
                    itc.reshape(N))
    lp = _epilogue(logits.reshape(N, 1), sp.reshape(N, 16))
    return (lp.reshape(N), gl)


# speedup vs baseline: 1.0659x; 1.0659x over previous
import functools

import jax
import jax.numpy as jnp
from jax import lax
from jax.experimental import pallas as pl
from jax.experimental.pallas import tpu as pltpu
from jax.experimental.pallas import tpu_sc as plsc

N = 1024
NSL = N // 16
SENT = 1 << 20


def _prologue_body(logits_ref, u_ref, adj_ref, gl_ref, explog_ref, deg_ref,
                   bits_ref, itc_ref):
    l = logits_ref[...]
    u = u_ref[...]
    gn = -jnp.log(-jnp.log(u + 1e-20) + 1e-20)
    gl_ref[...] = l + gn
    explog_ref[...] = jnp.exp(l)
    a = adj_ref[...]
    deg_ref[...] = jnp.sum(a, axis=1, keepdims=True)
    acc = jnp.zeros((32, N), jnp.int32)
    for b in range(32):
        acc = acc | (lax.slice(a, (b * 32, 0), (b * 32 + 32, N)) << b)
    bits_ref[...] = acc
    srl = lax.shift_right_logical
    x = acc - (srl(acc, 1) & 0x55555555)
    x = (x & 0x33333333) + (srl(x, 2) & 0x33333333)
    x = (x + srl(x, 4)) & 0x0F0F0F0F
    cnt = srl(x * 0x01010101, 24)
    itc_ref[...] = jnp.max(cnt, axis=0, keepdims=True)


_prologue = pl.pallas_call(
    _prologue_body,
    out_shape=(
        jax.ShapeDtypeStruct((8, 128), jnp.float32),
        jax.ShapeDtypeStruct((8, 128), jnp.float32),
        jax.ShapeDtypeStruct((N, 1), jnp.int32),
        jax.ShapeDtypeStruct((32, N), jnp.int32),
        jax.ShapeDtypeStruct((1, N), jnp.int32),
    ),
)


def _epilogue_body(logits_ref, sp_ref, out_ref):
    l = logits_ref[...]
    s = jnp.sum(sp_ref[...], axis=1, keepdims=True)
    out_ref[...] = 0.0 - jnp.log(s * jnp.exp(-l) + 1e-10)


_epilogue = pl.pallas_call(
    _epilogue_body,
    out_shape=jax.ShapeDtypeStruct((N, 1), jnp.float32),
)


def _make_sc_kernel():
    mesh = plsc.VectorSubcoreMesh(core_axis_name="c", subcore_axis_name="s")

    @functools.partial(
        pl.kernel,
        mesh=mesh,
        out_type=jax.ShapeDtypeStruct((N * 16,), jnp.float32),
        compiler_params=pltpu.CompilerParams(needs_layout_passes=False),
        scratch_types=[
            pltpu.VMEM((N,), jnp.float32),
            pltpu.VMEM((64,), jnp.float32),
            pltpu.VMEM((64,), jnp.float32),
            pltpu.VMEM((N,), jnp.int32),
            pltpu.VMEM((32, N), jnp.int32),
            pltpu.VMEM((N,), jnp.float32),
            pltpu.VMEM((N,), jnp.float32),
            pltpu.VMEM((N,), jnp.int32),
            pltpu.VMEM((N * 16,), jnp.float32),
        ],
    )
    def sc_kernel(gl_hbm, explog_hbm, deg_hbm, bits_hbm, itc_hbm, sp_hbm,
                  mval_v, bmax_v, bsum_v, deg_v, bits_v, gl_v,
                  explog_v, itc_v, sp_v):
        c = lax.axis_index("c")
        sub = lax.axis_index("s")

        def _perm(v, idx):
            return v.at[idx].get(mode="promise_in_bounds")

        def _splat(x):
            return x if getattr(x, "shape", ()) == (16,) else jnp.full(
                (16,), x, jnp.int32)

        def _bcast_last(v):
            return _perm(v, jnp.full((16,), 15, jnp.int32))

        @pl.when(jnp.logical_and(c == 0, sub == 0))
        def _():
            pltpu.sync_copy(gl_hbm, gl_v)
            pltpu.sync_copy(explog_hbm, explog_v)
            pltpu.sync_copy(deg_hbm, deg_v)
            pltpu.sync_copy(bits_hbm, bits_v)
            pltpu.sync_copy(itc_hbm, itc_v)
            iota = lax.broadcasted_iota(jnp.int32, (16,), 0)
            neg_inf = jnp.full((16,), -jnp.inf, jnp.float32)
            zeros_f = jnp.zeros((16,), jnp.float32)
            lane0 = iota == 0

            for k in range(NSL):
                sl = pl.ds(k * 16, 16)
                elig = deg_v[sl] == 0
                mval_v[sl] = jnp.where(elig, gl_v[sl], neg_inf)
            for v in range(4):
                bm = neg_inf
                bs = zeros_f
                for t in range(16):
                    sl = pl.ds(t * 64 + v * 16, 16)
                    mv = mval_v[sl]
                    bm = jnp.maximum(bm, mv)
                    bs = bs + jnp.where(mv > neg_inf, explog_v[sl], zeros_f)
                bmax_v[pl.ds(v * 16, 16)] = bm
                bsum_v[pl.ds(v * 16, 16)] = bs

            def step(_, carry):
                bb = bmax_v[pl.ds(0, 16)]
                bid = jnp.zeros((16,), jnp.int32)
                ss = bsum_v[pl.ds(0, 16)]
                for v in range(1, 4):
                    bv = bmax_v[pl.ds(v * 16, 16)]
                    m = bv > bb
                    bb = jnp.where(m, bv, bb)
                    bid = jnp.where(m, v, bid)
                    ss = ss + bsum_v[pl.ds(v * 16, 16)]
                r = _bcast_last(plsc.cummax(bb))
                lane = _splat(plsc.all_reduce_ffs(bb == r))
                vid = _perm(bid, lane)
                blk = vid * 16 + lane

                idxb = blk + 64 * iota
                bvals = plsc.load_gather(mval_v, [idxb])
                t = _splat(plsc.all_reduce_ffs(bvals == r))
                mi_v = blk + 64 * t

                plsc.store_scatter(sp_v, [mi_v * 16 + iota], ss)
                plsc.store_scatter(mval_v, [mi_v], neg_inf, mask=lane0)
                plsc.store_scatter(deg_v, [mi_v],
                                   jnp.full((16,), SENT, jnp.int32),
                                   mask=lane0)
                bv2 = jnp.where(iota == t, neg_inf, bvals)
                nbm = _bcast_last(plsc.cummax(bv2))
                plsc.store_scatter(bmax_v, [blk], nbm, mask=lane0)
                eg = plsc.load_gather(explog_v, [mi_v])
                plsc.addupdate_scatter(bsum_v, [blk], -eg, mask=lane0)

                w0 = plsc.load_gather(bits_v, [iota, mi_v])
                w1 = plsc.load_gather(bits_v, [iota + 16, mi_v])
                trip = jnp.max(plsc.load_gather(itc_v, [mi_v]))

                def adj_iter(wc):
                    nw = []
                    for wv, rowidx in ((wc[0], iota), (wc[1], iota + 16)):
                        act = wv != 0
                        y = wv & (-wv)
                        f = y.astype(jnp.float32)
                        e = ((lax.bitcast_convert_type(f, jnp.int32) >> 23)
                             & 255) - 127
                        i = jnp.where(act, e * 32 + rowidx, 0)
                        blkv = i & 63
                        g = plsc.load_gather(gl_v, [i], mask=act)
                        ee = plsc.load_gather(explog_v, [i], mask=act)
                        bm = plsc.load_gather(bmax_v, [blkv], mask=act)
                        od = plsc.load_gather(deg_v, [i], mask=act)
                        nd = od - 1
                        plsc.store_scatter(deg_v, [i], nd, mask=act)
                        trans = jnp.logical_and(act, nd == 0)
                        plsc.store_scatter(mval_v, [i], g, mask=trans)
                        plsc.store_scatter(bmax_v, [blkv], jnp.maximum(bm, g),
                                           mask=trans)
                        plsc.addupdate_scatter(bsum_v, [blkv], ee, mask=trans)
                        nw.append(wv & (wv - 1))
                    return tuple(nw)

                wc = adj_iter((w0, w1))

                def wcond(c):
                    return c[2] > 1

                def wbody(c):
                    nw = adj_iter((c[0], c[1]))
                    return (nw[0], nw[1], c[2] - 1)

                lax.while_loop(wcond, wbody, (wc[0], wc[1], trip))
                return carry

            lax.fori_loop(0, N, step, 0)
            pltpu.sync_copy(sp_v, sp_hbm)

    return sc_kernel


_sc_kernel = _make_sc_kernel()


def kernel(logits, adj_mat):
    u = jax.random.uniform(jax.random.key(42), logits.shape,
                           dtype=logits.dtype)
    l2 = logits.reshape(8, 128)
    gl2, explog2, deg2, bits, itc = _prologue(l2, u.reshape(8, 128), adj_mat)
    gl = gl2.reshape(N)
    sp = _sc_kernel(gl, explog2.reshape(N), deg2.reshape(N), bits,
                    itc.reshape(N))
    lp = _epilogue(logits.reshape(N, 1), sp.reshape(N, 16))
    return (lp.reshape(N), gl)


# speedup vs baseline: 1.0668x; 1.0008x over previous
import functools

import jax
import jax.numpy as jnp
from jax import lax
from jax.experimental import pallas as pl
from jax.experimental.pallas import tpu as pltpu
from jax.experimental.pallas import tpu_sc as plsc

N = 1024
NSL = N // 16
SENT = 1 << 20


def _prologue_body(logits_ref, u_ref, adj_ref, gl_ref, explog_ref, deg_ref,
                   bits_ref, itc_ref):
    l = logits_ref[...]
    u = u_ref[...]
    gn = -jnp.log(-jnp.log(u + 1e-20) + 1e-20)
    gl_ref[...] = l + gn
    explog_ref[...] = jnp.exp(l)
    a = adj_ref[...]
    deg_ref[...] = jnp.sum(a, axis=1, keepdims=True)
    acc = jnp.zeros((32, N), jnp.int32)
    for b in range(32):
        acc = acc | (lax.slice(a, (b * 32, 0), (b * 32 + 32, N)) << b)
    bits_ref[...] = acc
    srl = lax.shift_right_logical
    x = acc - (srl(acc, 1) & 0x55555555)
    x = (x & 0x33333333) + (srl(x, 2) & 0x33333333)
    x = (x + srl(x, 4)) & 0x0F0F0F0F
    cnt = srl(x * 0x01010101, 24)
    itc_ref[...] = jnp.max(cnt, axis=0, keepdims=True)


_prologue = pl.pallas_call(
    _prologue_body,
    out_shape=(
        jax.ShapeDtypeStruct((8, 128), jnp.float32),
        jax.ShapeDtypeStruct((8, 128), jnp.float32),
        jax.ShapeDtypeStruct((N, 1), jnp.int32),
        jax.ShapeDtypeStruct((32, N), jnp.int32),
        jax.ShapeDtypeStruct((1, N), jnp.int32),
    ),
)


def _epilogue_body(logits_ref, sp_ref, out_ref):
    l = logits_ref[...]
    s = jnp.sum(sp_ref[...], axis=1, keepdims=True)
    out_ref[...] = 0.0 - jnp.log(s * jnp.exp(-l) + 1e-10)


_epilogue = pl.pallas_call(
    _epilogue_body,
    out_shape=jax.ShapeDtypeStruct((N, 1), jnp.float32),
)


def _make_sc_kernel():
    mesh = plsc.VectorSubcoreMesh(core_axis_name="c", subcore_axis_name="s")

    @functools.partial(
        pl.kernel,
        mesh=mesh,
        out_type=jax.ShapeDtypeStruct((N * 16,), jnp.float32),
        compiler_params=pltpu.CompilerParams(needs_layout_passes=False),
        scratch_types=[
            pltpu.VMEM((N,), jnp.float32),
            pltpu.VMEM((64,), jnp.float32),
            pltpu.VMEM((64,), jnp.float32),
            pltpu.VMEM((N,), jnp.int32),
            pltpu.VMEM((32, N), jnp.int32),
            pltpu.VMEM((N,), jnp.float32),
            pltpu.VMEM((N,), jnp.float32),
            pltpu.VMEM((N,), jnp.int32),
            pltpu.VMEM((N * 16,), jnp.float32),
        ],
    )
    def sc_kernel(gl_hbm, explog_hbm, deg_hbm, bits_hbm, itc_hbm, sp_hbm,
                  mval_v, bmax_v, bsum_v, deg_v, bits_v, gl_v,
                  explog_v, itc_v, sp_v):
        c = lax.axis_index("c")
        sub = lax.axis_index("s")

        def _perm(v, idx):
            return v.at[idx].get(mode="promise_in_bounds")

        def _splat(x):
            return x if getattr(x, "shape", ()) == (16,) else jnp.full(
                (16,), x, jnp.int32)

        def _bcast_last(v):
            return _perm(v, jnp.full((16,), 15, jnp.int32))

        @pl.when(jnp.logical_and(c == 0, sub == 0))
        def _():
            pltpu.sync_copy(gl_hbm, gl_v)
            pltpu.sync_copy(explog_hbm, explog_v)
            pltpu.sync_copy(deg_hbm, deg_v)
            pltpu.sync_copy(bits_hbm, bits_v)
            pltpu.sync_copy(itc_hbm, itc_v)
            iota = lax.broadcasted_iota(jnp.int32, (16,), 0)
            neg_inf = jnp.full((16,), -jnp.inf, jnp.float32)
            zeros_f = jnp.zeros((16,), jnp.float32)
            lane0 = iota == 0

            for k in range(NSL):
                sl = pl.ds(k * 16, 16)
                elig = deg_v[sl] == 0
                mval_v[sl] = jnp.where(elig, gl_v[sl], neg_inf)
            for v in range(4):
                bm = neg_inf
                bs = zeros_f
                for t in range(16):
                    sl = pl.ds(t * 64 + v * 16, 16)
                    mv = mval_v[sl]
                    bm = jnp.maximum(bm, mv)
                    bs = bs + jnp.where(mv > neg_inf, explog_v[sl], zeros_f)
                bmax_v[pl.ds(v * 16, 16)] = bm
                bsum_v[pl.ds(v * 16, 16)] = bs

            def step(_, carry):
                b0 = bmax_v[pl.ds(0, 16)]
                b1 = bmax_v[pl.ds(16, 16)]
                b2 = bmax_v[pl.ds(32, 16)]
                b3 = bmax_v[pl.ds(48, 16)]
                m01 = b1 > b0
                v01 = jnp.where(m01, b1, b0)
                i01 = jnp.where(m01, 1, 0)
                m23 = b3 > b2
                v23 = jnp.where(m23, b3, b2)
                i23 = jnp.where(m23, 3, 2)
                mf = v23 > v01
                bb = jnp.where(mf, v23, v01)
                bid = jnp.where(mf, i23, i01)
                ss = ((bsum_v[pl.ds(0, 16)] + bsum_v[pl.ds(16, 16)])
                      + (bsum_v[pl.ds(32, 16)] + bsum_v[pl.ds(48, 16)]))
                r = _bcast_last(plsc.cummax(bb))
                lane = _splat(plsc.all_reduce_ffs(bb == r))
                vid = _perm(bid, lane)
                blk = vid * 16 + lane

                idxb = blk + 64 * iota
                bvals = plsc.load_gather(mval_v, [idxb])
                t = _splat(plsc.all_reduce_ffs(bvals == r))
                mi_v = blk + 64 * t

                plsc.store_scatter(sp_v, [mi_v * 16 + iota], ss)
                plsc.store_scatter(mval_v, [mi_v], neg_inf, mask=lane0)
                bv2 = jnp.where(iota == t, neg_inf, bvals)
                nbm = _bcast_last(plsc.cummax(bv2))
                plsc.store_scatter(bmax_v, [blk], nbm, mask=lane0)
                eg = plsc.load_gather(explog_v, [mi_v])
                plsc.addupdate_scatter(bsum_v, [blk], -eg, mask=lane0)

                w0 = plsc.load_gather(bits_v, [iota, mi_v])
                w1 = plsc.load_gather(bits_v, [iota + 16, mi_v])
                trip = jnp.max(plsc.load_gather(itc_v, [mi_v]))

                def adj_iter(wc):
                    nw = []
                    for wv, rowidx in ((wc[0], iota), (wc[1], iota + 16)):
                        act = wv != 0
                        y = wv & (-wv)
                        f = y.astype(jnp.float32)
                        e = ((lax.bitcast_convert_type(f, jnp.int32) >> 23)
                             & 255) - 127
                        i = jnp.where(act, e * 32 + rowidx, 0)
                        blkv = i & 63
                        g = plsc.load_gather(gl_v, [i], mask=act)
                        ee = plsc.load_gather(explog_v, [i], mask=act)
                        bm = plsc.load_gather(bmax_v, [blkv], mask=act)
                        od = plsc.load_gather(deg_v, [i], mask=act)
                        nd = od - 1
                        plsc.store_scatter(deg_v, [i], nd, mask=act)
                        trans = jnp.logical_and(act, nd == 0)
                        plsc.store_scatter(mval_v, [i], g, mask=trans)
                        plsc.store_scatter(bmax_v, [blkv], jnp.maximum(bm, g),
                                           mask=trans)
                        plsc.addupdate_scatter(bsum_v, [blkv], ee, mask=trans)
                        nw.append(wv & (wv - 1))
                    return tuple(nw)

                wc = adj_iter((w0, w1))

                def wcond(c):
                    return c[2] > 1

                def wbody(c):
                    nw = adj_iter((c[0], c[1]))
                    return (nw[0], nw[1], c[2] - 1)

                lax.while_loop(wcond, wbody, (wc[0], wc[1], trip))
                return carry

            lax.fori_loop(0, N, step, 0)
            pltpu.sync_copy(sp_v, sp_hbm)

    return sc_kernel


_sc_kernel = _make_sc_kernel()


def kernel(logits, adj_mat):
    u = jax.random.uniform(jax.random.key(42), logits.shape,
                           dtype=logits.dtype)
    l2 = logits.reshape(8, 128)
    gl2, explog2, deg2, bits, itc = _prologue(l2, u.reshape(8, 128), adj_mat)
    gl = gl2.reshape(N)
    sp = _sc_kernel(gl, explog2.reshape(N), deg2.reshape(N), bits,
                    itc.reshape(N))
    lp = _epilogue(logits.reshape(N, 1), sp.reshape(N, 16))
    return (lp.reshape(N), gl)
